# Initial kernel scaffold; baseline (speedup 1.0000x reference)
import jax
import jax.numpy as jnp
from jax.experimental import pallas as pl


def kernel(x, edge_index, W1, b1, W2, b2, W3, b3):
    raise NotImplementedError("write your pallas kernel here")


# speedup vs baseline: 42.1751x; 42.1751x over previous
import functools

import jax
import jax.numpy as jnp
from jax import lax
from jax.experimental import pallas as pl
from jax.experimental.pallas import tpu as pltpu
from jax.experimental.pallas import tpu_sc as plsc

N = 10000
D = 128
E = 320000

NS = 16
LANES = 16
NPT = 640
NP = NS * NPT
NB = NPT // LANES
EPT = E // NS
C = 128
NCHUNK = 157
EPT_PAD = NCHUNK * C
EPAD = EPT_PAD - EPT


def _rsqrt_newton(x):
    i = lax.bitcast_convert_type(x, jnp.int32)
    y = lax.bitcast_convert_type(jnp.int32(0x5F3759DF) - (i >> 1), jnp.float32)
    for _ in range(3):
        y = y * (1.5 - 0.5 * x * y * y)
    return y


def _sc_body(src_hbm, dst_hbm, h1t_hbm, params_hbm, out_hbm,
             src_v, dst_v, params_v, msg_v, msg2_v, ones_v,
             nodef_v, cols_v, dinv_v, vec_v,
             deg_sh, g1c0_sh, g1c1_sh, g1c2_sh,
             s1c0_sh, s1c1_sh, s1c2_sh, g2_sh, s2_sh):
    g1c = [g1c0_sh, g1c1_sh, g1c2_sh]
    s1c = [s1c0_sh, s1c1_sh, s1c2_sh]
    sid = lax.axis_index("s")
    base = sid * NPT

    pltpu.sync_copy(src_hbm.at[pl.ds(sid * EPT_PAD, EPT_PAD)], src_v)
    pltpu.sync_copy(dst_hbm.at[pl.ds(sid * EPT_PAD, EPT_PAD)], dst_v)
    pltpu.sync_copy(params_hbm, params_v)

    for b in range(C // LANES):
        ones_v[pl.ds(b * LANES, LANES)] = jnp.full((LANES,), 1.0, jnp.float32)

    @pl.loop(0, NB)
    def _(b):
        nodef_v[pl.ds(b * LANES, LANES)] = jnp.full((LANES,), 1.0, jnp.float32)
    pltpu.sync_copy(nodef_v, deg_sh.at[pl.ds(base, NPT)])

    plsc.subcore_barrier()

    @pl.loop(0, NCHUNK)
    def _(j):
        pltpu.sync_copy(ones_v, deg_sh.at[dst_v.at[pl.ds(j * C, C)]],
                        add=True)

    plsc.subcore_barrier()

    pltpu.sync_copy(deg_sh.at[pl.ds(base, NPT)], nodef_v)
    for c in range(3):
        pltpu.sync_copy(h1t_hbm.at[pl.ds(c * NP + base, NPT)],
                        cols_v.at[pl.ds(c * NPT, NPT)])

    @pl.loop(0, NB)
    def _(b):
        bs = pl.ds(b * LANES, LANES)
        deg16 = nodef_v[bs]
        dinv16 = _rsqrt_newton(deg16)
        dinv_v[bs] = dinv16
        for c in range(3):
            cs = pl.ds(c * NPT + b * LANES, LANES)
            cols_v[cs] = cols_v[cs] * dinv16

    for c in range(3):
        pltpu.sync_copy(cols_v.at[pl.ds(c * NPT, NPT)],
                        g1c[c].at[pl.ds(base, NPT)])
        pltpu.sync_copy(cols_v.at[pl.ds(c * NPT, NPT)],
                        s1c[c].at[pl.ds(base, NPT)])

    plsc.subcore_barrier()

    @pl.loop(0, NCHUNK)
    def _(j):
        for c in range(3):
            pltpu.sync_copy(g1c[c].at[src_v.at[pl.ds(j * C, C)]], msg_v)
            pltpu.sync_copy(msg_v, s1c[c].at[dst_v.at[pl.ds(j * C, C)]],
                            add=True)

    plsc.subcore_barrier()

    for c in range(3):
        pltpu.sync_copy(s1c[c].at[pl.ds(base, NPT)],
                        cols_v.at[pl.ds(c * NPT, NPT)])

    @pl.loop(0, NB)
    def _(b):
        bs = pl.ds(b * LANES, LANES)
        dinv16 = dinv_v[bs]
        h2 = jnp.full((LANES,), 0.0, jnp.float32)
        for c in range(3):
            cs = pl.ds(c * NPT + b * LANES, LANES)
            b1c = params_v[pl.ds(c * LANES, LANES)]
            w2c = params_v[pl.ds((3 + c) * LANES, LANES)]
            a1c = jnp.maximum(dinv16 * cols_v[cs] + b1c, 0.0)
            h2 = h2 + a1c * w2c
        vec_v[bs] = dinv16 * h2

    pltpu.sync_copy(vec_v, g2_sh.at[pl.ds(base, NPT)])
    pltpu.sync_copy(vec_v, s2_sh.at[pl.ds(base, NPT)])

    plsc.subcore_barrier()

    @pl.loop(0, NCHUNK)
    def _(j):
        pltpu.sync_copy(g2_sh.at[src_v.at[pl.ds(j * C, C)]], msg2_v)
        pltpu.sync_copy(msg2_v, s2_sh.at[dst_v.at[pl.ds(j * C, C)]],
                        add=True)

    plsc.subcore_barrier()

    pltpu.sync_copy(s2_sh.at[pl.ds(base, NPT)], nodef_v)

    @pl.loop(0, NB)
    def _(b):
        bs = pl.ds(b * LANES, LANES)
        b2v = params_v[pl.ds(6 * LANES, LANES)]
        vec_v[bs] = dinv_v[bs] * nodef_v[bs] + b2v

    pltpu.sync_copy(vec_v, out_hbm.at[pl.ds(base, NPT)])


@jax.jit
def _sc_pass(srcp, dstp, h1, params):
    mesh = plsc.VectorSubcoreMesh(
        core_axis_name="c", subcore_axis_name="s", num_cores=1)
    f = pl.kernel(
        _sc_body,
        out_type=jax.ShapeDtypeStruct((NP,), jnp.float32),
        mesh=mesh,
        scratch_types=[
            pltpu.VMEM((EPT_PAD,), jnp.int32),
            pltpu.VMEM((EPT_PAD,), jnp.int32),
            pltpu.VMEM((8 * LANES,), jnp.float32),
            pltpu.VMEM((C,), jnp.float32),
            pltpu.VMEM((C,), jnp.float32),
            pltpu.VMEM((C,), jnp.float32),
            pltpu.VMEM((NPT,), jnp.float32),
            pltpu.VMEM((3 * NPT,), jnp.float32),
            pltpu.VMEM((NPT,), jnp.float32),
            pltpu.VMEM((NPT,), jnp.float32),
            pltpu.VMEM_SHARED((NP,), jnp.float32),
            pltpu.VMEM_SHARED((NP,), jnp.float32),
            pltpu.VMEM_SHARED((NP,), jnp.float32),
            pltpu.VMEM_SHARED((NP,), jnp.float32),
            pltpu.VMEM_SHARED((NP,), jnp.float32),
            pltpu.VMEM_SHARED((NP,), jnp.float32),
            pltpu.VMEM_SHARED((NP,), jnp.float32),
            pltpu.VMEM_SHARED((NP,), jnp.float32),
            pltpu.VMEM_SHARED((NP,), jnp.float32),
        ],
    )
    return f(srcp, dstp, h1, params)


def _mm_body(x_ref, w_ref, o_ref):
    o_ref[...] = jnp.dot(x_ref[...], w_ref[...],
                         preferred_element_type=jnp.float32)


@jax.jit
def _tc_linear(xp, W1):
    return pl.pallas_call(
        _mm_body,
        grid=(NP // 1024,),
        in_specs=[
            pl.BlockSpec((1024, D), lambda i: (i, 0)),
            pl.BlockSpec((D, 3), lambda i: (0, 0)),
        ],
        out_specs=pl.BlockSpec((1024, 3), lambda i: (i, 0)),
        out_shape=jax.ShapeDtypeStruct((NP, 3), jnp.float32),
    )(xp, W1)


def _head_body(a_ref, w_ref, b_ref, o_ref):
    logits = jnp.dot(a_ref[...], w_ref[...],
                     preferred_element_type=jnp.float32) + b_ref[...]
    m = jnp.max(logits, axis=1, keepdims=True)
    e = jnp.exp(logits - m)
    lse = m + jnp.log(jnp.sum(e, axis=1, keepdims=True))
    o_ref[...] = logits - lse


@jax.jit
def _tc_head(a, W3, b3):
    return pl.pallas_call(
        _head_body,
        out_shape=jax.ShapeDtypeStruct((N // 5, 2), jnp.float32),
    )(a, W3, b3.reshape(1, 2))


def kernel(x, edge_index, W1, b1, W2, b2, W3, b3):
    xp = jnp.pad(x, ((0, NP - N), (0, 0)))
    h1t = _tc_linear(xp, W1).T.reshape(-1)

    pad_rows = (N + 16 + jnp.arange(EPAD, dtype=jnp.int32))
    src2 = edge_index[0].reshape(NS, EPT)
    dst2 = edge_index[1].reshape(NS, EPT)
    pad2 = jnp.broadcast_to(pad_rows, (NS, EPAD))
    srcp = jnp.concatenate([src2, pad2], axis=1).reshape(-1)
    dstp = jnp.concatenate([dst2, pad2], axis=1).reshape(-1)

    params = jnp.concatenate(
        [b1, W2[:, 0], b2, jnp.zeros((1,), jnp.float32)])
    params = jnp.repeat(params, LANES)

    out2 = _sc_pass(srcp, dstp, h1t, params)
    a = out2[:N].reshape(N // 5, 5)
    return _tc_head(a, W3, b3)


# speedup vs baseline: 59.9237x; 1.4208x over previous
import jax
import jax.numpy as jnp
from jax import lax
from jax.experimental import pallas as pl
from jax.experimental.pallas import tpu as pltpu
from jax.experimental.pallas import tpu_sc as plsc

N = 10000
D = 128
E = 320000

NC = 2
NS = 16
NW = NC * NS
LANES = 16
NPT = 640
NP = NS * NPT
NB = NPT // LANES
EPW = E // NW
C = 128
NCHUNK = (EPW + C - 1) // C
EPW_PAD = NCHUNK * C
EPAD = EPW_PAD - EPW


def _mesh():
    return plsc.VectorSubcoreMesh(core_axis_name="c", subcore_axis_name="s")


def _worker(base_only=False):
    cid = lax.axis_index("c")
    sid = lax.axis_index("s")
    return cid, sid


def _deg_body(dst_hbm, out_hbm, dst_v, ones_v, nodef_v, deg_sh):
    cid, sid = _worker()
    base = sid * NPT
    ebase = (cid * NS + sid) * EPW_PAD

    pltpu.sync_copy(dst_hbm.at[pl.ds(ebase, EPW_PAD)], dst_v)
    for b in range(C // LANES):
        ones_v[pl.ds(b * LANES, LANES)] = jnp.full((LANES,), 1.0, jnp.float32)

    @pl.loop(0, NB)
    def _(b):
        nodef_v[pl.ds(b * LANES, LANES)] = jnp.full((LANES,), 0.0, jnp.float32)
    pltpu.sync_copy(nodef_v, deg_sh.at[pl.ds(base, NPT)])

    plsc.subcore_barrier()

    @pl.loop(0, NCHUNK)
    def _(j):
        pltpu.sync_copy(ones_v, deg_sh.at[dst_v.at[pl.ds(j * C, C)]],
                        add=True)

    plsc.subcore_barrier()
    pltpu.sync_copy(deg_sh.at[pl.ds(base, NPT)],
                    out_hbm.at[pl.ds(cid * NP + base, NPT)])


@jax.jit
def _sc_deg(dstp):
    f = pl.kernel(
        _deg_body,
        out_type=jax.ShapeDtypeStruct((NC * NP,), jnp.float32),
        mesh=_mesh(),
        scratch_types=[
            pltpu.VMEM((EPW_PAD,), jnp.int32),
            pltpu.VMEM((C,), jnp.float32),
            pltpu.VMEM((NPT,), jnp.float32),
            pltpu.VMEM_SHARED((NP,), jnp.float32),
        ],
    )
    return f(dstp)


def _l1_body(src_hbm, dst_hbm, g1t_hbm, out_hbm, src_v, dst_v, msg_v,
             g1c0_sh, g1c1_sh, g1c2_sh, s1c0_sh, s1c1_sh, s1c2_sh):
    g1c = [g1c0_sh, g1c1_sh, g1c2_sh]
    s1c = [s1c0_sh, s1c1_sh, s1c2_sh]
    cid, sid = _worker()
    base = sid * NPT
    ebase = (cid * NS + sid) * EPW_PAD

    pltpu.sync_copy(src_hbm.at[pl.ds(ebase, EPW_PAD)], src_v)
    pltpu.sync_copy(dst_hbm.at[pl.ds(ebase, EPW_PAD)], dst_v)
    for c in range(3):
        pltpu.sync_copy(g1t_hbm.at[pl.ds(c * NP + base, NPT)],
                        g1c[c].at[pl.ds(base, NPT)])
        pltpu.sync_copy(g1t_hbm.at[pl.ds(c * NP + base, NPT)],
                        s1c[c].at[pl.ds(base, NPT)])

    plsc.subcore_barrier()

    @pl.loop(0, NCHUNK)
    def _(j):
        for c in range(3):
            pltpu.sync_copy(g1c[c].at[src_v.at[pl.ds(j * C, C)]], msg_v)
            pltpu.sync_copy(msg_v, s1c[c].at[dst_v.at[pl.ds(j * C, C)]],
                            add=True)

    plsc.subcore_barrier()
    for c in range(3):
        pltpu.sync_copy(
            s1c[c].at[pl.ds(base, NPT)],
            out_hbm.at[pl.ds(c * NC * NP + cid * NP + base, NPT)])


@jax.jit
def _sc_l1(srcp, dstp, g1t):
    f = pl.kernel(
        _l1_body,
        out_type=jax.ShapeDtypeStruct((3 * NC * NP,), jnp.float32),
        mesh=_mesh(),
        scratch_types=[
            pltpu.VMEM((EPW_PAD,), jnp.int32),
            pltpu.VMEM((EPW_PAD,), jnp.int32),
            pltpu.VMEM((C,), jnp.float32),
            pltpu.VMEM_SHARED((NP,), jnp.float32),
            pltpu.VMEM_SHARED((NP,), jnp.float32),
            pltpu.VMEM_SHARED((NP,), jnp.float32),
            pltpu.VMEM_SHARED((NP,), jnp.float32),
            pltpu.VMEM_SHARED((NP,), jnp.float32),
            pltpu.VMEM_SHARED((NP,), jnp.float32),
        ],
    )
    return f(srcp, dstp, g1t)


def _l2_body(src_hbm, dst_hbm, g2_hbm, out_hbm, src_v, dst_v, msg_v,
             g2_sh, s2_sh):
    cid, sid = _worker()
    base = sid * NPT
    ebase = (cid * NS + sid) * EPW_PAD

    pltpu.sync_copy(src_hbm.at[pl.ds(ebase, EPW_PAD)], src_v)
    pltpu.sync_copy(dst_hbm.at[pl.ds(ebase, EPW_PAD)], dst_v)
    pltpu.sync_copy(g2_hbm.at[pl.ds(base, NPT)],
                    g2_sh.at[pl.ds(base, NPT)])
    pltpu.sync_copy(g2_hbm.at[pl.ds(base, NPT)],
                    s2_sh.at[pl.ds(base, NPT)])

    plsc.subcore_barrier()

    @pl.loop(0, NCHUNK)
    def _(j):
        pltpu.sync_copy(g2_sh.at[src_v.at[pl.ds(j * C, C)]], msg_v)
        pltpu.sync_copy(msg_v, s2_sh.at[dst_v.at[pl.ds(j * C, C)]],
                        add=True)

    plsc.subcore_barrier()
    pltpu.sync_copy(s2_sh.at[pl.ds(base, NPT)],
                    out_hbm.at[pl.ds(cid * NP + base, NPT)])


@jax.jit
def _sc_l2(srcp, dstp, g2):
    f = pl.kernel(
        _l2_body,
        out_type=jax.ShapeDtypeStruct((NC * NP,), jnp.float32),
        mesh=_mesh(),
        scratch_types=[
            pltpu.VMEM((EPW_PAD,), jnp.int32),
            pltpu.VMEM((EPW_PAD,), jnp.int32),
            pltpu.VMEM((C,), jnp.float32),
            pltpu.VMEM_SHARED((NP,), jnp.float32),
            pltpu.VMEM_SHARED((NP,), jnp.float32),
        ],
    )
    return f(srcp, dstp, g2)


def _tc1_body(x_ref, w_ref, da_ref, db_ref, g1t_ref, dinv_ref):
    h1 = jnp.dot(x_ref[...], w_ref[...], preferred_element_type=jnp.float32)
    deg = 1.0 + da_ref[...] + db_ref[...]
    dinv = lax.rsqrt(deg)
    g1t_ref[...] = h1.T * dinv
    dinv_ref[...] = dinv


@jax.jit
def _tc_prep(xp, W1, dega, degb):
    blk = 1024
    return pl.pallas_call(
        _tc1_body,
        grid=(NP // blk,),
        in_specs=[
            pl.BlockSpec((blk, D), lambda i: (i, 0)),
            pl.BlockSpec((D, 4), lambda i: (0, 0)),
            pl.BlockSpec((1, blk), lambda i: (0, i)),
            pl.BlockSpec((1, blk), lambda i: (0, i)),
        ],
        out_specs=[
            pl.BlockSpec((4, blk), lambda i: (0, i)),
            pl.BlockSpec((1, blk), lambda i: (0, i)),
        ],
        out_shape=[
            jax.ShapeDtypeStruct((4, NP), jnp.float32),
            jax.ShapeDtypeStruct((1, NP), jnp.float32),
        ],
    )(xp, W1, dega, degb)


def _tc2_body(sa_ref, sb_ref, g1t_ref, dinv_ref, b1_ref, w2_ref, g2_ref):
    dinv = dinv_ref[0, :]
    h2 = jnp.zeros_like(dinv)
    for c in range(3):
        s1c = sa_ref[c, :] + sb_ref[c, :] - g1t_ref[c, :]
        a1c = jnp.maximum(dinv * s1c + b1_ref[0, c], 0.0)
        h2 = h2 + a1c * w2_ref[c, 0]
    g2_ref[0, :] = dinv * h2


@jax.jit
def _tc_mid(s1a, s1b, g1t, dinv, b1, W2):
    blk = 1024
    return pl.pallas_call(
        _tc2_body,
        grid=(NP // blk,),
        in_specs=[
            pl.BlockSpec((3, blk), lambda i: (0, i)),
            pl.BlockSpec((3, blk), lambda i: (0, i)),
            pl.BlockSpec((4, blk), lambda i: (0, i)),
            pl.BlockSpec((1, blk), lambda i: (0, i)),
            pl.BlockSpec((1, 3), lambda i: (0, 0)),
            pl.BlockSpec((3, 1), lambda i: (0, 0)),
        ],
        out_specs=pl.BlockSpec((1, blk), lambda i: (0, i)),
        out_shape=jax.ShapeDtypeStruct((1, NP), jnp.float32),
    )(s1a, s1b, g1t, dinv, b1, W2)


def _tc3_body(sa_ref, sb_ref, g2_ref, dinv_ref, b2_ref, w3_ref, b3_ref,
              o_ref):
    out2 = dinv_ref[...] * (sa_ref[...] + sb_ref[...] - g2_ref[...])
    out2 = out2 + b2_ref[0, 0]
    logits = jnp.dot(out2, w3_ref[...], preferred_element_type=jnp.float32)
    logits = logits + b3_ref[...]
    m = jnp.max(logits, axis=1, keepdims=True)
    e = jnp.exp(logits - m)
    lse = m + jnp.log(jnp.sum(e, axis=1, keepdims=True))
    o_ref[...] = logits - lse


@jax.jit
def _tc_head(s2a, s2b, g2, dinv, b2, W3, b3):
    return pl.pallas_call(
        _tc3_body,
        out_shape=jax.ShapeDtypeStruct((N // 5, 2), jnp.float32),
    )(s2a, s2b, g2, dinv, b2.reshape(1, 1), W3, b3.reshape(1, 2))


def kernel(x, edge_index, W1, b1, W2, b2, W3, b3):
    pad_rows = N + 16 + jnp.arange(EPAD, dtype=jnp.int32)
    src2 = edge_index[0].reshape(NW, EPW)
    dst2 = edge_index[1].reshape(NW, EPW)
    pad2 = jnp.broadcast_to(pad_rows, (NW, EPAD))
    srcp = jnp.concatenate([src2, pad2], axis=1).reshape(-1)
    dstp = jnp.concatenate([dst2, pad2], axis=1).reshape(-1)

    degp = _sc_deg(dstp)
    dega = degp[:NP].reshape(1, NP)
    degb = degp[NP:].reshape(1, NP)

    xp = jnp.pad(x, ((0, NP - N), (0, 0)))
    W1p = jnp.pad(W1, ((0, 0), (0, 1)))
    g1t, dinv = _tc_prep(xp, W1p, dega, degb)

    s1p = _sc_l1(srcp, dstp, g1t[:3].reshape(-1))
    s1r = s1p.reshape(3, NC, NP)
    g2 = _tc_mid(s1r[:, 0], s1r[:, 1], g1t, dinv, b1.reshape(1, 3), W2)

    s2p = _sc_l2(srcp, dstp, g2.reshape(-1))

    def head_view(v):
        return v[:N].reshape(N // 5, 5)

    return _tc_head(head_view(s2p[:NP]), head_view(s2p[NP:]),
                    head_view(g2.reshape(-1)), head_view(dinv.reshape(-1)),
                    b2, W3, b3)


# speedup vs baseline: 63.6050x; 1.0614x over previous
import jax
import jax.numpy as jnp
from jax import lax
from jax.experimental import pallas as pl
from jax.experimental.pallas import tpu as pltpu
from jax.experimental.pallas import tpu_sc as plsc

N = 10000
D = 128
E = 320000

NC = 2
NS = 16
NW = NC * NS
LANES = 16
NPT = 640
NP = NS * NPT
NB = NPT // LANES
EPW = E // NW
C = 128
NCHUNK = (EPW + C - 1) // C
EPW_PAD = NCHUNK * C
EPAD = EPW_PAD - EPW


def _mesh():
    return plsc.VectorSubcoreMesh(core_axis_name="c", subcore_axis_name="s")


def _rsqrt_newton(x):
    i = lax.bitcast_convert_type(x, jnp.int32)
    y = lax.bitcast_convert_type(jnp.int32(0x5F3759DF) - (i >> 1), jnp.float32)
    for _ in range(3):
        y = y * (1.5 - 0.5 * x * y * y)
    return y


def _deg_body(dst_hbm, out_hbm, dst_v, ones_v, nodef_v, deg_sh):
    cid = lax.axis_index("c")
    sid = lax.axis_index("s")
    base = sid * NPT
    ebase = (cid * NS + sid) * EPW_PAD

    pltpu.sync_copy(dst_hbm.at[pl.ds(ebase, EPW_PAD)], dst_v)
    for b in range(C // LANES):
        ones_v[pl.ds(b * LANES, LANES)] = jnp.full((LANES,), 1.0, jnp.float32)

    @pl.loop(0, NB)
    def _(b):
        nodef_v[pl.ds(b * LANES, LANES)] = jnp.full((LANES,), 0.0, jnp.float32)
    pltpu.sync_copy(nodef_v, deg_sh.at[pl.ds(base, NPT)])

    plsc.subcore_barrier()

    @pl.loop(0, NCHUNK)
    def _(j):
        pltpu.sync_copy(ones_v, deg_sh.at[dst_v.at[pl.ds(j * C, C)]],
                        add=True)

    plsc.subcore_barrier()
    pltpu.sync_copy(deg_sh.at[pl.ds(base, NPT)],
                    out_hbm.at[pl.ds(cid * NP + base, NPT)])


@jax.jit
def _sc_deg(dstp):
    f = pl.kernel(
        _deg_body,
        out_type=jax.ShapeDtypeStruct((NC * NP,), jnp.float32),
        mesh=_mesh(),
        scratch_types=[
            pltpu.VMEM((EPW_PAD,), jnp.int32),
            pltpu.VMEM((C,), jnp.float32),
            pltpu.VMEM((NPT,), jnp.float32),
            pltpu.VMEM_SHARED((NP,), jnp.float32),
        ],
    )
    return f(dstp)


def _l1_body(src_hbm, dst_hbm, h1t_hbm, degp_hbm,
             s1out_hbm, dinv_out_hbm,
             src_v, dst_v, msg_v, dega_v, degb_v, dinv_v, col_v,
             g1c0_sh, g1c1_sh, g1c2_sh, s1c0_sh, s1c1_sh, s1c2_sh):
    g1c = [g1c0_sh, g1c1_sh, g1c2_sh]
    s1c = [s1c0_sh, s1c1_sh, s1c2_sh]
    cid = lax.axis_index("c")
    sid = lax.axis_index("s")
    base = sid * NPT
    ebase = (cid * NS + sid) * EPW_PAD

    pltpu.sync_copy(src_hbm.at[pl.ds(ebase, EPW_PAD)], src_v)
    pltpu.sync_copy(dst_hbm.at[pl.ds(ebase, EPW_PAD)], dst_v)
    pltpu.sync_copy(degp_hbm.at[pl.ds(base, NPT)], dega_v)
    pltpu.sync_copy(degp_hbm.at[pl.ds(NP + base, NPT)], degb_v)

    @pl.loop(0, NB)
    def _(b):
        bs = pl.ds(b * LANES, LANES)
        deg16 = 1.0 + dega_v[bs] + degb_v[bs]
        dinv_v[bs] = _rsqrt_newton(deg16)

    for c in range(3):
        pltpu.sync_copy(h1t_hbm.at[pl.ds(c * NP + base, NPT)], col_v)

        @pl.loop(0, NB)
        def _(b):
            bs = pl.ds(b * LANES, LANES)
            col_v[bs] = col_v[bs] * dinv_v[bs]
        pltpu.sync_copy(col_v, g1c[c].at[pl.ds(base, NPT)])

        @pl.when(cid == 0)
        def _():
            pltpu.sync_copy(col_v, s1c[c].at[pl.ds(base, NPT)])

        @pl.when(cid != 0)
        def _():
            @pl.loop(0, NB)
            def _(b):
                col_v[pl.ds(b * LANES, LANES)] = jnp.full(
                    (LANES,), 0.0, jnp.float32)
            pltpu.sync_copy(col_v, s1c[c].at[pl.ds(base, NPT)])

    plsc.subcore_barrier()

    @pl.loop(0, NCHUNK)
    def _(j):
        for c in range(3):
            pltpu.sync_copy(g1c[c].at[src_v.at[pl.ds(j * C, C)]], msg_v)
            pltpu.sync_copy(msg_v, s1c[c].at[dst_v.at[pl.ds(j * C, C)]],
                            add=True)

    plsc.subcore_barrier()
    for c in range(3):
        pltpu.sync_copy(
            s1c[c].at[pl.ds(base, NPT)],
            s1out_hbm.at[pl.ds(c * NC * NP + cid * NP + base, NPT)])

    @pl.when(cid == 0)
    def _():
        pltpu.sync_copy(dinv_v, dinv_out_hbm.at[pl.ds(base, NPT)])


@jax.jit
def _sc_l1(srcp, dstp, h1t3, degp):
    f = pl.kernel(
        _l1_body,
        out_type=[
            jax.ShapeDtypeStruct((3 * NC * NP,), jnp.float32),
            jax.ShapeDtypeStruct((NP,), jnp.float32),
        ],
        mesh=_mesh(),
        scratch_types=[
            pltpu.VMEM((EPW_PAD,), jnp.int32),
            pltpu.VMEM((EPW_PAD,), jnp.int32),
            pltpu.VMEM((C,), jnp.float32),
            pltpu.VMEM((NPT,), jnp.float32),
            pltpu.VMEM((NPT,), jnp.float32),
            pltpu.VMEM((NPT,), jnp.float32),
            pltpu.VMEM((NPT,), jnp.float32),
            pltpu.VMEM_SHARED((NP,), jnp.float32),
            pltpu.VMEM_SHARED((NP,), jnp.float32),
            pltpu.VMEM_SHARED((NP,), jnp.float32),
            pltpu.VMEM_SHARED((NP,), jnp.float32),
            pltpu.VMEM_SHARED((NP,), jnp.float32),
            pltpu.VMEM_SHARED((NP,), jnp.float32),
        ],
    )
    return f(srcp, dstp, h1t3, degp)


def _l2_body(src_hbm, dst_hbm, s1_hbm, dinv_hbm, params_hbm, out_hbm,
             src_v, dst_v, msg_v, params_v, dinv_v, colsa_v, colsb_v,
             vec_v, g2_sh, s2_sh):
    cid = lax.axis_index("c")
    sid = lax.axis_index("s")
    base = sid * NPT
    ebase = (cid * NS + sid) * EPW_PAD

    pltpu.sync_copy(src_hbm.at[pl.ds(ebase, EPW_PAD)], src_v)
    pltpu.sync_copy(dst_hbm.at[pl.ds(ebase, EPW_PAD)], dst_v)
    pltpu.sync_copy(params_hbm, params_v)
    pltpu.sync_copy(dinv_hbm.at[pl.ds(base, NPT)], dinv_v)
    for c in range(3):
        pltpu.sync_copy(s1_hbm.at[pl.ds(c * NC * NP + base, NPT)],
                        colsa_v.at[pl.ds(c * NPT, NPT)])
        pltpu.sync_copy(s1_hbm.at[pl.ds(c * NC * NP + NP + base, NPT)],
                        colsb_v.at[pl.ds(c * NPT, NPT)])

    @pl.loop(0, NB)
    def _(b):
        bs = pl.ds(b * LANES, LANES)
        dinv16 = dinv_v[bs]
        h2 = jnp.full((LANES,), 0.0, jnp.float32)
        for c in range(3):
            cs = pl.ds(c * NPT + b * LANES, LANES)
            b1c = params_v[pl.ds(c * LANES, LANES)]
            w2c = params_v[pl.ds((3 + c) * LANES, LANES)]
            s1c16 = colsa_v[cs] + colsb_v[cs]
            a1c = jnp.maximum(dinv16 * s1c16 + b1c, 0.0)
            h2 = h2 + a1c * w2c
        vec_v[bs] = dinv16 * h2

    pltpu.sync_copy(vec_v, g2_sh.at[pl.ds(base, NPT)])

    @pl.when(cid != 0)
    def _():
        @pl.loop(0, NB)
        def _(b):
            vec_v[pl.ds(b * LANES, LANES)] = jnp.full(
                (LANES,), 0.0, jnp.float32)
    pltpu.sync_copy(vec_v, s2_sh.at[pl.ds(base, NPT)])

    plsc.subcore_barrier()

    @pl.loop(0, NCHUNK)
    def _(j):
        pltpu.sync_copy(g2_sh.at[src_v.at[pl.ds(j * C, C)]], msg_v)
        pltpu.sync_copy(msg_v, s2_sh.at[dst_v.at[pl.ds(j * C, C)]],
                        add=True)

    plsc.subcore_barrier()
    pltpu.sync_copy(s2_sh.at[pl.ds(base, NPT)],
                    out_hbm.at[pl.ds(cid * NP + base, NPT)])


@jax.jit
def _sc_l2(srcp, dstp, s1p, dinv, params):
    f = pl.kernel(
        _l2_body,
        out_type=jax.ShapeDtypeStruct((NC * NP,), jnp.float32),
        mesh=_mesh(),
        scratch_types=[
            pltpu.VMEM((EPW_PAD,), jnp.int32),
            pltpu.VMEM((EPW_PAD,), jnp.int32),
            pltpu.VMEM((C,), jnp.float32),
            pltpu.VMEM((8 * LANES,), jnp.float32),
            pltpu.VMEM((NPT,), jnp.float32),
            pltpu.VMEM((3 * NPT,), jnp.float32),
            pltpu.VMEM((3 * NPT,), jnp.float32),
            pltpu.VMEM((NPT,), jnp.float32),
            pltpu.VMEM_SHARED((NP,), jnp.float32),
            pltpu.VMEM_SHARED((NP,), jnp.float32),
        ],
    )
    return f(srcp, dstp, s1p, dinv, params)


def _mm_body(x_ref, w_ref, h1t_ref):
    h1 = jnp.dot(x_ref[...], w_ref[...], preferred_element_type=jnp.float32)
    h1t_ref[...] = h1.T


@jax.jit
def _tc_mm(xp, W1p):
    blk = 1024
    return pl.pallas_call(
        _mm_body,
        grid=(NP // blk,),
        in_specs=[
            pl.BlockSpec((blk, D), lambda i: (i, 0)),
            pl.BlockSpec((D, 4), lambda i: (0, 0)),
        ],
        out_specs=pl.BlockSpec((4, blk), lambda i: (0, i)),
        out_shape=jax.ShapeDtypeStruct((4, NP), jnp.float32),
    )(xp, W1p)


def _head_body(sa_ref, sb_ref, dinv_ref, b2_ref, w3_ref, b3_ref, o_ref):
    out2 = dinv_ref[...] * (sa_ref[...] + sb_ref[...]) + b2_ref[0, 0]
    logits = jnp.dot(out2, w3_ref[...], preferred_element_type=jnp.float32)
    logits = logits + b3_ref[...]
    m = jnp.max(logits, axis=1, keepdims=True)
    e = jnp.exp(logits - m)
    lse = m + jnp.log(jnp.sum(e, axis=1, keepdims=True))
    o_ref[...] = logits - lse


@jax.jit
def _tc_head(s2a, s2b, dinv, b2, W3, b3):
    return pl.pallas_call(
        _head_body,
        out_shape=jax.ShapeDtypeStruct((N // 5, 2), jnp.float32),
    )(s2a, s2b, dinv, b2.reshape(1, 1), W3, b3.reshape(1, 2))


def kernel(x, edge_index, W1, b1, W2, b2, W3, b3):
    pad_rows = N + 16 + jnp.arange(EPAD, dtype=jnp.int32)
    src2 = edge_index[0].reshape(NW, EPW)
    dst2 = edge_index[1].reshape(NW, EPW)
    pad2 = jnp.broadcast_to(pad_rows, (NW, EPAD))
    srcp = jnp.concatenate([src2, pad2], axis=1).reshape(-1)
    dstp = jnp.concatenate([dst2, pad2], axis=1).reshape(-1)

    xp = jnp.pad(x, ((0, NP - N), (0, 0)))
    W1p = jnp.pad(W1, ((0, 0), (0, 1)))
    h1t = _tc_mm(xp, W1p)

    degp = _sc_deg(dstp)
    s1p, dinv = _sc_l1(srcp, dstp, h1t[:3].reshape(-1), degp)

    params = jnp.concatenate(
        [jnp.repeat(b1, LANES), jnp.repeat(W2[:, 0], LANES),
         jnp.zeros((2 * LANES,), jnp.float32)])
    s2p = _sc_l2(srcp, dstp, s1p, dinv, params)

    def hv(v):
        return v[:N].reshape(N // 5, 5)

    return _tc_head(hv(s2p[:NP]), hv(s2p[NP:]), hv(dinv), b2, W3, b3)


# speedup vs baseline: 92.9107x; 1.4607x over previous
import jax
import jax.numpy as jnp
from jax import lax
from jax.experimental import pallas as pl
from jax.experimental.pallas import tpu as pltpu
from jax.experimental.pallas import tpu_sc as plsc

N = 10000
D = 128
E = 320000

NC = 2
NS = 16
NW = NC * NS
LANES = 16
NPT = 640
NP = NS * NPT
NB = NPT // LANES
EPW = E // NW
C = 128
NCHUNK = 80
EPW_PAD = NCHUNK * C
EPAD = EPW_PAD - EPW
KPC = C // LANES


def _mesh():
    return plsc.VectorSubcoreMesh(core_axis_name="c", subcore_axis_name="s")


def _zero_fill(ref):
    @pl.loop(0, NB)
    def _(b):
        ref[pl.ds(b * LANES, LANES)] = jnp.full((LANES,), 0.0, jnp.float32)


def _deg_body(dst_hbm, out_hbm, dst_v, ones_v, nodef_v, deg_sh,
              sem0, sem1):
    cid = lax.axis_index("c")
    sid = lax.axis_index("s")
    base = sid * NPT
    ebase = (cid * NS + sid) * EPW_PAD

    pltpu.sync_copy(dst_hbm.at[pl.ds(ebase, EPW_PAD)], dst_v)
    for b in range(C // LANES):
        ones_v[pl.ds(b * LANES, LANES)] = jnp.full((LANES,), 1.0, jnp.float32)
    _zero_fill(nodef_v)
    pltpu.sync_copy(nodef_v, deg_sh.at[pl.ds(base, NPT)])

    plsc.subcore_barrier()

    def dref(j):
        return deg_sh.at[dst_v.at[pl.ds(j * C, C)]]

    pltpu.async_copy(ones_v, dref(0), sem0, add=True)

    @pl.loop(0, NCHUNK // 2)
    def _(jj):
        j0 = 2 * jj
        j1 = j0 + 1

        @pl.when(jj > 0)
        def _():
            pltpu.make_async_copy(ones_v, dref(j1 - 2), sem1).wait()
        pltpu.async_copy(ones_v, dref(j1), sem1, add=True)

        @pl.when(jj < NCHUNK // 2 - 1)
        def _():
            pltpu.make_async_copy(ones_v, dref(j0), sem0).wait()
            pltpu.async_copy(ones_v, dref(j0 + 2), sem0, add=True)

    pltpu.make_async_copy(ones_v, dref(NCHUNK - 2), sem0).wait()
    pltpu.make_async_copy(ones_v, dref(NCHUNK - 1), sem1).wait()

    plsc.subcore_barrier()
    pltpu.sync_copy(deg_sh.at[pl.ds(base, NPT)],
                    out_hbm.at[pl.ds(cid * NP + base, NPT)])


@jax.jit
def _sc_deg(dstp):
    f = pl.kernel(
        _deg_body,
        out_type=jax.ShapeDtypeStruct((NC * NP,), jnp.float32),
        mesh=_mesh(),
        scratch_types=[
            pltpu.VMEM((EPW_PAD,), jnp.int32),
            pltpu.VMEM((C,), jnp.float32),
            pltpu.VMEM((NPT,), jnp.float32),
            pltpu.VMEM_SHARED((NP,), jnp.float32),
            pltpu.SemaphoreType.DMA,
            pltpu.SemaphoreType.DMA,
        ],
    )
    return f(dstp)


def _l1_body(src_hbm, dst_hbm, g1t_hbm, out_hbm,
             src_v, dst_v, msg_v, col_v,
             g1l0_v, g1l1_v, g1l2_v,
             s1c0_sh, s1c1_sh, s1c2_sh, sem0, sem1):
    g1l = [g1l0_v, g1l1_v, g1l2_v]
    s1c = [s1c0_sh, s1c1_sh, s1c2_sh]
    sems = [sem0, sem1]
    cid = lax.axis_index("c")
    sid = lax.axis_index("s")
    base = sid * NPT
    ebase = (cid * NS + sid) * EPW_PAD

    pltpu.sync_copy(src_hbm.at[pl.ds(ebase, EPW_PAD)], src_v)
    pltpu.sync_copy(dst_hbm.at[pl.ds(ebase, EPW_PAD)], dst_v)
    for c in range(3):
        pltpu.sync_copy(g1t_hbm.at[pl.ds(c * NP, NP)], g1l[c])

    @pl.when(cid == 0)
    def _():
        for c in range(3):
            pltpu.sync_copy(g1l[c].at[pl.ds(base, NPT)],
                            s1c[c].at[pl.ds(base, NPT)])

    @pl.when(cid != 0)
    def _():
        _zero_fill(col_v)
        for c in range(3):
            pltpu.sync_copy(col_v, s1c[c].at[pl.ds(base, NPT)])

    plsc.subcore_barrier()

    def build(j, p):
        for k in range(KPC):
            idx16 = src_v[pl.ds(j * C + k * LANES, LANES)]
            for c in range(3):
                msg_v[pl.ds((p * 3 + c) * C + k * LANES, LANES)] = (
                    plsc.load_gather(g1l[c], [idx16]))

    def mb(p, c):
        return msg_v.at[pl.ds((p * 3 + c) * C, C)]

    def sref(j, c):
        return s1c[c].at[dst_v.at[pl.ds(j * C, C)]]

    def sstart(j, p):
        for c in range(3):
            pltpu.async_copy(mb(p, c), sref(j, c), sems[p], add=True)

    def swait(j, p):
        for c in range(3):
            pltpu.make_async_copy(mb(p, c), sref(j, c), sems[p]).wait()

    build(0, 0)
    sstart(0, 0)

    @pl.loop(0, NCHUNK // 2)
    def _(jj):
        j0 = 2 * jj
        j1 = j0 + 1

        @pl.when(jj > 0)
        def _():
            swait(j1 - 2, 1)
        build(j1, 1)
        sstart(j1, 1)

        @pl.when(jj < NCHUNK // 2 - 1)
        def _():
            swait(j0, 0)
            build(j0 + 2, 0)
            sstart(j0 + 2, 0)

    swait(NCHUNK - 2, 0)
    swait(NCHUNK - 1, 1)

    plsc.subcore_barrier()
    for c in range(3):
        pltpu.sync_copy(
            s1c[c].at[pl.ds(base, NPT)],
            out_hbm.at[pl.ds(c * NC * NP + cid * NP + base, NPT)])


@jax.jit
def _sc_l1(srcp, dstp, g1t3):
    f = pl.kernel(
        _l1_body,
        out_type=jax.ShapeDtypeStruct((3 * NC * NP,), jnp.float32),
        mesh=_mesh(),
        compiler_params=pltpu.CompilerParams(needs_layout_passes=False),
        scratch_types=[
            pltpu.VMEM((EPW_PAD,), jnp.int32),
            pltpu.VMEM((EPW_PAD,), jnp.int32),
            pltpu.VMEM((2 * 3 * C,), jnp.float32),
            pltpu.VMEM((NPT,), jnp.float32),
            pltpu.VMEM((NP,), jnp.float32),
            pltpu.VMEM((NP,), jnp.float32),
            pltpu.VMEM((NP,), jnp.float32),
            pltpu.VMEM_SHARED((NP,), jnp.float32),
            pltpu.VMEM_SHARED((NP,), jnp.float32),
            pltpu.VMEM_SHARED((NP,), jnp.float32),
            pltpu.SemaphoreType.DMA,
            pltpu.SemaphoreType.DMA,
        ],
    )
    return f(srcp, dstp, g1t3)


def _l2_body(src_hbm, dst_hbm, g2_hbm, out_hbm,
             src_v, dst_v, msg_v, col_v, g2l_v, s2_sh, sem0, sem1):
    sems = [sem0, sem1]
    cid = lax.axis_index("c")
    sid = lax.axis_index("s")
    base = sid * NPT
    ebase = (cid * NS + sid) * EPW_PAD

    pltpu.sync_copy(src_hbm.at[pl.ds(ebase, EPW_PAD)], src_v)
    pltpu.sync_copy(dst_hbm.at[pl.ds(ebase, EPW_PAD)], dst_v)
    pltpu.sync_copy(g2_hbm, g2l_v)

    @pl.when(cid == 0)
    def _():
        pltpu.sync_copy(g2l_v.at[pl.ds(base, NPT)],
                        s2_sh.at[pl.ds(base, NPT)])

    @pl.when(cid != 0)
    def _():
        _zero_fill(col_v)
        pltpu.sync_copy(col_v, s2_sh.at[pl.ds(base, NPT)])

    plsc.subcore_barrier()

    def build(j, p):
        for k in range(KPC):
            idx16 = src_v[pl.ds(j * C + k * LANES, LANES)]
            msg_v[pl.ds(p * C + k * LANES, LANES)] = (
                plsc.load_gather(g2l_v, [idx16]))

    def mb(p):
        return msg_v.at[pl.ds(p * C, C)]

    def sref(j):
        return s2_sh.at[dst_v.at[pl.ds(j * C, C)]]

    build(0, 0)
    pltpu.async_copy(mb(0), sref(0), sem0, add=True)

    @pl.loop(0, NCHUNK // 2)
    def _(jj):
        j0 = 2 * jj
        j1 = j0 + 1

        @pl.when(jj > 0)
        def _():
            pltpu.make_async_copy(mb(1), sref(j1 - 2), sem1).wait()
        build(j1, 1)
        pltpu.async_copy(mb(1), sref(j1), sem1, add=True)

        @pl.when(jj < NCHUNK // 2 - 1)
        def _():
            pltpu.make_async_copy(mb(0), sref(j0), sem0).wait()
            build(j0 + 2, 0)
            pltpu.async_copy(mb(0), sref(j0 + 2), sem0, add=True)

    pltpu.make_async_copy(mb(0), sref(NCHUNK - 2), sem0).wait()
    pltpu.make_async_copy(mb(1), sref(NCHUNK - 1), sem1).wait()

    plsc.subcore_barrier()
    pltpu.sync_copy(s2_sh.at[pl.ds(base, NPT)],
                    out_hbm.at[pl.ds(cid * NP + base, NPT)])


@jax.jit
def _sc_l2(srcp, dstp, g2):
    f = pl.kernel(
        _l2_body,
        out_type=jax.ShapeDtypeStruct((NC * NP,), jnp.float32),
        mesh=_mesh(),
        compiler_params=pltpu.CompilerParams(needs_layout_passes=False),
        scratch_types=[
            pltpu.VMEM((EPW_PAD,), jnp.int32),
            pltpu.VMEM((EPW_PAD,), jnp.int32),
            pltpu.VMEM((2 * C,), jnp.float32),
            pltpu.VMEM((NPT,), jnp.float32),
            pltpu.VMEM((NP,), jnp.float32),
            pltpu.VMEM_SHARED((NP,), jnp.float32),
            pltpu.SemaphoreType.DMA,
            pltpu.SemaphoreType.DMA,
        ],
    )
    return f(srcp, dstp, g2)


def _tc1_body(x_ref, w_ref, da_ref, db_ref, g1t_ref, dinv_ref):
    h1 = jnp.dot(x_ref[...], w_ref[...], preferred_element_type=jnp.float32)
    deg = 1.0 + da_ref[...] + db_ref[...]
    dinv = lax.rsqrt(deg)
    g1t_ref[...] = h1.T * dinv
    dinv_ref[...] = dinv


@jax.jit
def _tc_prep(xp, W1, dega, degb):
    blk = 1024
    return pl.pallas_call(
        _tc1_body,
        grid=(NP // blk,),
        in_specs=[
            pl.BlockSpec((blk, D), lambda i: (i, 0)),
            pl.BlockSpec((D, 4), lambda i: (0, 0)),
            pl.BlockSpec((1, blk), lambda i: (0, i)),
            pl.BlockSpec((1, blk), lambda i: (0, i)),
        ],
        out_specs=[
            pl.BlockSpec((4, blk), lambda i: (0, i)),
            pl.BlockSpec((1, blk), lambda i: (0, i)),
        ],
        out_shape=[
            jax.ShapeDtypeStruct((4, NP), jnp.float32),
            jax.ShapeDtypeStruct((1, NP), jnp.float32),
        ],
    )(xp, W1, dega, degb)


def _tc2_body(sa_ref, sb_ref, dinv_ref, b1_ref, w2_ref, g2_ref):
    dinv = dinv_ref[0, :]
    h2 = jnp.zeros_like(dinv)
    for c in range(3):
        s1c = sa_ref[c, :] + sb_ref[c, :]
        a1c = jnp.maximum(dinv * s1c + b1_ref[0, c], 0.0)
        h2 = h2 + a1c * w2_ref[c, 0]
    g2_ref[0, :] = dinv * h2


@jax.jit
def _tc_mid(s1a, s1b, dinv, b1, W2):
    blk = 1024
    return pl.pallas_call(
        _tc2_body,
        grid=(NP // blk,),
        in_specs=[
            pl.BlockSpec((3, blk), lambda i: (0, i)),
            pl.BlockSpec((3, blk), lambda i: (0, i)),
            pl.BlockSpec((1, blk), lambda i: (0, i)),
            pl.BlockSpec((1, 3), lambda i: (0, 0)),
            pl.BlockSpec((3, 1), lambda i: (0, 0)),
        ],
        out_specs=pl.BlockSpec((1, blk), lambda i: (0, i)),
        out_shape=jax.ShapeDtypeStruct((1, NP), jnp.float32),
    )(s1a, s1b, dinv, b1, W2)


def _tc3_body(sa_ref, sb_ref, dinv_ref, b2_ref, w3_ref, b3_ref, o_ref):
    out2 = dinv_ref[...] * (sa_ref[...] + sb_ref[...]) + b2_ref[0, 0]
    logits = jnp.dot(out2, w3_ref[...], preferred_element_type=jnp.float32)
    logits = logits + b3_ref[...]
    m = jnp.max(logits, axis=1, keepdims=True)
    e = jnp.exp(logits - m)
    lse = m + jnp.log(jnp.sum(e, axis=1, keepdims=True))
    o_ref[...] = logits - lse


@jax.jit
def _tc_head(s2a, s2b, dinv, b2, W3, b3):
    return pl.pallas_call(
        _tc3_body,
        out_shape=jax.ShapeDtypeStruct((N // 5, 2), jnp.float32),
    )(s2a, s2b, dinv, b2.reshape(1, 1), W3, b3.reshape(1, 2))


def kernel(x, edge_index, W1, b1, W2, b2, W3, b3):
    pad_rows = N + jnp.arange(EPAD, dtype=jnp.int32)
    src2 = edge_index[0].reshape(NW, EPW)
    dst2 = edge_index[1].reshape(NW, EPW)
    pad2 = jnp.broadcast_to(pad_rows, (NW, EPAD))
    srcp = jnp.concatenate([src2, pad2], axis=1).reshape(-1)
    dstp = jnp.concatenate([dst2, pad2], axis=1).reshape(-1)

    degp = _sc_deg(dstp)
    dega = degp[:NP].reshape(1, NP)
    degb = degp[NP:].reshape(1, NP)

    xp = jnp.pad(x, ((0, NP - N), (0, 0)))
    W1p = jnp.pad(W1, ((0, 0), (0, 1)))
    g1t, dinv = _tc_prep(xp, W1p, dega, degb)

    s1p = _sc_l1(srcp, dstp, g1t[:3].reshape(-1))
    s1r = s1p.reshape(3, NC, NP)
    g2 = _tc_mid(s1r[:, 0], s1r[:, 1], dinv, b1.reshape(1, 3), W2)

    s2p = _sc_l2(srcp, dstp, g2.reshape(-1))

    def hv(v):
        return v[:N].reshape(N // 5, 5)

    return _tc_head(hv(s2p[:NP]), hv(s2p[NP:]), hv(dinv.reshape(-1)),
                    b2, W3, b3)


# speedup vs baseline: 93.8077x; 1.0097x over previous
import jax
import jax.numpy as jnp
from jax import lax
from jax.experimental import pallas as pl
from jax.experimental.pallas import tpu as pltpu
from jax.experimental.pallas import tpu_sc as plsc

N = 10000
D = 128
E = 320000

NC = 2
NS = 16
NW = NC * NS
LANES = 16
NPT = 640
NP = NS * NPT
NB = NPT // LANES
EPW = E // NW
C = 128
NCHUNK = 80
EPW_PAD = NCHUNK * C
EPAD = EPW_PAD - EPW
KPC = C // LANES


def _mesh():
    return plsc.VectorSubcoreMesh(core_axis_name="c", subcore_axis_name="s")


def _zero_fill(ref):
    @pl.loop(0, NB)
    def _(b):
        ref[pl.ds(b * LANES, LANES)] = jnp.full((LANES,), 0.0, jnp.float32)


def _deg_body(dst_hbm, out_hbm, dst_v, ones_v, nodef_v, deg_sh,
              sem0, sem1):
    cid = lax.axis_index("c")
    sid = lax.axis_index("s")
    base = sid * NPT
    ebase = (cid * NS + sid) * EPW_PAD

    pltpu.sync_copy(dst_hbm.at[pl.ds(ebase, EPW_PAD)], dst_v)
    for b in range(C // LANES):
        ones_v[pl.ds(b * LANES, LANES)] = jnp.full((LANES,), 1.0, jnp.float32)
    _zero_fill(nodef_v)
    pltpu.sync_copy(nodef_v, deg_sh.at[pl.ds(base, NPT)])

    plsc.subcore_barrier()

    def dref(j):
        return deg_sh.at[dst_v.at[pl.ds(j * C, C)]]

    pltpu.async_copy(ones_v, dref(0), sem0, add=True)

    @pl.loop(0, NCHUNK // 2)
    def _(jj):
        j0 = 2 * jj
        j1 = j0 + 1

        @pl.when(jj > 0)
        def _():
            pltpu.make_async_copy(ones_v, dref(j1 - 2), sem1).wait()
        pltpu.async_copy(ones_v, dref(j1), sem1, add=True)

        @pl.when(jj < NCHUNK // 2 - 1)
        def _():
            pltpu.make_async_copy(ones_v, dref(j0), sem0).wait()
            pltpu.async_copy(ones_v, dref(j0 + 2), sem0, add=True)

    pltpu.make_async_copy(ones_v, dref(NCHUNK - 2), sem0).wait()
    pltpu.make_async_copy(ones_v, dref(NCHUNK - 1), sem1).wait()

    plsc.subcore_barrier()
    pltpu.sync_copy(deg_sh.at[pl.ds(base, NPT)],
                    out_hbm.at[pl.ds(cid * NP + base, NPT)])


@jax.jit
def _sc_deg(dstp):
    f = pl.kernel(
        _deg_body,
        out_type=jax.ShapeDtypeStruct((NC * NP,), jnp.float32),
        mesh=_mesh(),
        scratch_types=[
            pltpu.VMEM((EPW_PAD,), jnp.int32),
            pltpu.VMEM((C,), jnp.float32),
            pltpu.VMEM((NPT,), jnp.float32),
            pltpu.VMEM_SHARED((NP,), jnp.float32),
            pltpu.SemaphoreType.DMA,
            pltpu.SemaphoreType.DMA,
        ],
    )
    return f(dstp)


def _l1_body(src_hbm, dst_hbm, g1t_hbm, out_hbm,
             src_v, dst_v, msg_v, col_v,
             g1l0_v, g1l1_v, g1l2_v,
             s1c0_sh, s1c1_sh, s1c2_sh, sem0, sem1):
    g1l = [g1l0_v, g1l1_v, g1l2_v]
    s1c = [s1c0_sh, s1c1_sh, s1c2_sh]
    sems = [sem0, sem1]
    cid = lax.axis_index("c")
    sid = lax.axis_index("s")
    base = sid * NPT
    ebase = (cid * NS + sid) * EPW_PAD

    pltpu.sync_copy(src_hbm.at[pl.ds(ebase, EPW_PAD)], src_v)
    pltpu.sync_copy(dst_hbm.at[pl.ds(ebase, EPW_PAD)], dst_v)
    for c in range(3):
        pltpu.sync_copy(g1t_hbm.at[pl.ds(c * NP, NP)], g1l[c])

    @pl.when(cid == 0)
    def _():
        for c in range(3):
            pltpu.sync_copy(g1l[c].at[pl.ds(base, NPT)],
                            s1c[c].at[pl.ds(base, NPT)])

    @pl.when(cid != 0)
    def _():
        _zero_fill(col_v)
        for c in range(3):
            pltpu.sync_copy(col_v, s1c[c].at[pl.ds(base, NPT)])

    plsc.subcore_barrier()

    def build(j, p):
        for k in range(KPC):
            idx16 = src_v[pl.ds(j * C + k * LANES, LANES)]
            for c in range(3):
                msg_v[pl.ds((p * 3 + c) * C + k * LANES, LANES)] = (
                    plsc.load_gather(g1l[c], [idx16]))

    def mb(p, c):
        return msg_v.at[pl.ds((p * 3 + c) * C, C)]

    def sref(j, c):
        return s1c[c].at[dst_v.at[pl.ds(j * C, C)]]

    def sstart(j, p):
        for c in range(3):
            pltpu.async_copy(mb(p, c), sref(j, c), sems[p], add=True)

    def swait(j, p):
        for c in range(3):
            pltpu.make_async_copy(mb(p, c), sref(j, c), sems[p]).wait()

    build(0, 0)
    sstart(0, 0)

    @pl.loop(0, NCHUNK // 2)
    def _(jj):
        j0 = 2 * jj
        j1 = j0 + 1

        @pl.when(jj > 0)
        def _():
            swait(j1 - 2, 1)
        build(j1, 1)
        sstart(j1, 1)

        @pl.when(jj < NCHUNK // 2 - 1)
        def _():
            swait(j0, 0)
            build(j0 + 2, 0)
            sstart(j0 + 2, 0)

    swait(NCHUNK - 2, 0)
    swait(NCHUNK - 1, 1)

    plsc.subcore_barrier()
    for c in range(3):
        pltpu.sync_copy(
            s1c[c].at[pl.ds(base, NPT)],
            out_hbm.at[pl.ds(c * NC * NP + cid * NP + base, NPT)])


@jax.jit
def _sc_l1(srcp, dstp, g1t3):
    f = pl.kernel(
        _l1_body,
        out_type=jax.ShapeDtypeStruct((3 * NC * NP,), jnp.float32),
        mesh=_mesh(),
        compiler_params=pltpu.CompilerParams(needs_layout_passes=False),
        scratch_types=[
            pltpu.VMEM((EPW_PAD,), jnp.int32),
            pltpu.VMEM((EPW_PAD,), jnp.int32),
            pltpu.VMEM((2 * 3 * C,), jnp.float32),
            pltpu.VMEM((NPT,), jnp.float32),
            pltpu.VMEM((NP,), jnp.float32),
            pltpu.VMEM((NP,), jnp.float32),
            pltpu.VMEM((NP,), jnp.float32),
            pltpu.VMEM_SHARED((NP,), jnp.float32),
            pltpu.VMEM_SHARED((NP,), jnp.float32),
            pltpu.VMEM_SHARED((NP,), jnp.float32),
            pltpu.SemaphoreType.DMA,
            pltpu.SemaphoreType.DMA,
        ],
    )
    return f(srcp, dstp, g1t3)


def _l2_body(src_hbm, dst_hbm, s1_hbm, dinv_hbm, params_hbm, out_hbm,
             src_v, dst_v, msg_v, col_v, g2l_v, params_v, dinv_v,
             colsa_v, colsb_v, vec_v, g2st_hbm, s2_sh, sem0, sem1):
    sems = [sem0, sem1]
    cid = lax.axis_index("c")
    sid = lax.axis_index("s")
    base = sid * NPT
    ebase = (cid * NS + sid) * EPW_PAD

    pltpu.sync_copy(src_hbm.at[pl.ds(ebase, EPW_PAD)], src_v)
    pltpu.sync_copy(dst_hbm.at[pl.ds(ebase, EPW_PAD)], dst_v)
    pltpu.sync_copy(params_hbm, params_v)
    pltpu.sync_copy(dinv_hbm.at[pl.ds(base, NPT)], dinv_v)
    for c in range(3):
        pltpu.sync_copy(s1_hbm.at[pl.ds(c * NC * NP + base, NPT)],
                        colsa_v.at[pl.ds(c * NPT, NPT)])
        pltpu.sync_copy(s1_hbm.at[pl.ds(c * NC * NP + NP + base, NPT)],
                        colsb_v.at[pl.ds(c * NPT, NPT)])

    @pl.loop(0, NB)
    def _(b):
        bs = pl.ds(b * LANES, LANES)
        dinv16 = dinv_v[bs]
        h2 = jnp.full((LANES,), 0.0, jnp.float32)
        for c in range(3):
            cs = pl.ds(c * NPT + b * LANES, LANES)
            b1c = params_v[pl.ds(c * LANES, LANES)]
            w2c = params_v[pl.ds((3 + c) * LANES, LANES)]
            a1c = jnp.maximum(dinv16 * (colsa_v[cs] + colsb_v[cs]) + b1c, 0.0)
            h2 = h2 + a1c * w2c
        vec_v[bs] = dinv16 * h2

    pltpu.sync_copy(vec_v, g2st_hbm.at[pl.ds(cid * NP + base, NPT)])

    @pl.when(cid == 0)
    def _():
        pltpu.sync_copy(vec_v, s2_sh.at[pl.ds(base, NPT)])

    @pl.when(cid != 0)
    def _():
        _zero_fill(col_v)
        pltpu.sync_copy(col_v, s2_sh.at[pl.ds(base, NPT)])

    plsc.subcore_barrier()
    pltpu.sync_copy(g2st_hbm.at[pl.ds(cid * NP, NP)], g2l_v)

    def build(j, p):
        for k in range(KPC):
            idx16 = src_v[pl.ds(j * C + k * LANES, LANES)]
            msg_v[pl.ds(p * C + k * LANES, LANES)] = (
                plsc.load_gather(g2l_v, [idx16]))

    def mb(p):
        return msg_v.at[pl.ds(p * C, C)]

    def sref(j):
        return s2_sh.at[dst_v.at[pl.ds(j * C, C)]]

    build(0, 0)
    pltpu.async_copy(mb(0), sref(0), sem0, add=True)

    @pl.loop(0, NCHUNK // 2)
    def _(jj):
        j0 = 2 * jj
        j1 = j0 + 1

        @pl.when(jj > 0)
        def _():
            pltpu.make_async_copy(mb(1), sref(j1 - 2), sem1).wait()
        build(j1, 1)
        pltpu.async_copy(mb(1), sref(j1), sem1, add=True)

        @pl.when(jj < NCHUNK // 2 - 1)
        def _():
            pltpu.make_async_copy(mb(0), sref(j0), sem0).wait()
            build(j0 + 2, 0)
            pltpu.async_copy(mb(0), sref(j0 + 2), sem0, add=True)

    pltpu.make_async_copy(mb(0), sref(NCHUNK - 2), sem0).wait()
    pltpu.make_async_copy(mb(1), sref(NCHUNK - 1), sem1).wait()

    plsc.subcore_barrier()
    pltpu.sync_copy(s2_sh.at[pl.ds(base, NPT)],
                    out_hbm.at[pl.ds(cid * NP + base, NPT)])


@jax.jit
def _sc_l2(srcp, dstp, s1p, dinv, params):
    f = pl.kernel(
        _l2_body,
        out_type=jax.ShapeDtypeStruct((NC * NP,), jnp.float32),
        mesh=_mesh(),
        compiler_params=pltpu.CompilerParams(needs_layout_passes=False),
        scratch_types=[
            pltpu.VMEM((EPW_PAD,), jnp.int32),
            pltpu.VMEM((EPW_PAD,), jnp.int32),
            pltpu.VMEM((2 * C,), jnp.float32),
            pltpu.VMEM((NPT,), jnp.float32),
            pltpu.VMEM((NP,), jnp.float32),
            pltpu.VMEM((8 * LANES,), jnp.float32),
            pltpu.VMEM((NPT,), jnp.float32),
            pltpu.VMEM((3 * NPT,), jnp.float32),
            pltpu.VMEM((3 * NPT,), jnp.float32),
            pltpu.VMEM((NPT,), jnp.float32),
            pltpu.HBM((NC * NP,), jnp.float32),
            pltpu.VMEM_SHARED((NP,), jnp.float32),
            pltpu.SemaphoreType.DMA,
            pltpu.SemaphoreType.DMA,
        ],
    )
    return f(srcp, dstp, s1p, dinv, params)


def _tc1_body(x_ref, w_ref, da_ref, db_ref, g1t_ref, dinv_ref):
    h1 = jnp.dot(x_ref[...], w_ref[...], preferred_element_type=jnp.float32)
    deg = 1.0 + da_ref[...] + db_ref[...]
    dinv = lax.rsqrt(deg)
    g1t_ref[...] = h1.T * dinv
    dinv_ref[...] = dinv


@jax.jit
def _tc_prep(xp, W1, dega, degb):
    blk = 1024
    return pl.pallas_call(
        _tc1_body,
        grid=(NP // blk,),
        in_specs=[
            pl.BlockSpec((blk, D), lambda i: (i, 0)),
            pl.BlockSpec((D, 4), lambda i: (0, 0)),
            pl.BlockSpec((1, blk), lambda i: (0, i)),
            pl.BlockSpec((1, blk), lambda i: (0, i)),
        ],
        out_specs=[
            pl.BlockSpec((4, blk), lambda i: (0, i)),
            pl.BlockSpec((1, blk), lambda i: (0, i)),
        ],
        out_shape=[
            jax.ShapeDtypeStruct((4, NP), jnp.float32),
            jax.ShapeDtypeStruct((1, NP), jnp.float32),
        ],
    )(xp, W1, dega, degb)


def _tc2_body(sa_ref, sb_ref, dinv_ref, b1_ref, w2_ref, g2_ref):
    dinv = dinv_ref[0, :]
    h2 = jnp.zeros_like(dinv)
    for c in range(3):
        s1c = sa_ref[c, :] + sb_ref[c, :]
        a1c = jnp.maximum(dinv * s1c + b1_ref[0, c], 0.0)
        h2 = h2 + a1c * w2_ref[c, 0]
    g2_ref[0, :] = dinv * h2


@jax.jit
def _tc_mid(s1a, s1b, dinv, b1, W2):
    blk = 1024
    return pl.pallas_call(
        _tc2_body,
        grid=(NP // blk,),
        in_specs=[
            pl.BlockSpec((3, blk), lambda i: (0, i)),
            pl.BlockSpec((3, blk), lambda i: (0, i)),
            pl.BlockSpec((1, blk), lambda i: (0, i)),
            pl.BlockSpec((1, 3), lambda i: (0, 0)),
            pl.BlockSpec((3, 1), lambda i: (0, 0)),
        ],
        out_specs=pl.BlockSpec((1, blk), lambda i: (0, i)),
        out_shape=jax.ShapeDtypeStruct((1, NP), jnp.float32),
    )(s1a, s1b, dinv, b1, W2)


def _tc3_body(sa_ref, sb_ref, dinv_ref, b2_ref, w3_ref, b3_ref, o_ref):
    out2 = dinv_ref[...] * (sa_ref[...] + sb_ref[...]) + b2_ref[0, 0]
    logits = jnp.dot(out2, w3_ref[...], preferred_element_type=jnp.float32)
    logits = logits + b3_ref[...]
    m = jnp.max(logits, axis=1, keepdims=True)
    e = jnp.exp(logits - m)
    lse = m + jnp.log(jnp.sum(e, axis=1, keepdims=True))
    o_ref[...] = logits - lse


@jax.jit
def _tc_head(s2a, s2b, dinv, b2, W3, b3):
    return pl.pallas_call(
        _tc3_body,
        out_shape=jax.ShapeDtypeStruct((N // 5, 2), jnp.float32),
    )(s2a, s2b, dinv, b2.reshape(1, 1), W3, b3.reshape(1, 2))


def kernel(x, edge_index, W1, b1, W2, b2, W3, b3):
    pad_rows = N + jnp.arange(EPAD, dtype=jnp.int32)
    src2 = edge_index[0].reshape(NW, EPW)
    dst2 = edge_index[1].reshape(NW, EPW)
    pad2 = jnp.broadcast_to(pad_rows, (NW, EPAD))
    srcp = jnp.concatenate([src2, pad2], axis=1).reshape(-1)
    dstp = jnp.concatenate([dst2, pad2], axis=1).reshape(-1)

    degp = _sc_deg(dstp)
    dega = degp[:NP].reshape(1, NP)
    degb = degp[NP:].reshape(1, NP)

    xp = jnp.pad(x, ((0, NP - N), (0, 0)))
    W1p = jnp.pad(W1, ((0, 0), (0, 1)))
    g1t, dinv = _tc_prep(xp, W1p, dega, degb)

    s1p = _sc_l1(srcp, dstp, g1t[:3].reshape(-1))

    params = jnp.concatenate(
        [jnp.repeat(b1, LANES), jnp.repeat(W2[:, 0], LANES),
         jnp.zeros((2 * LANES,), jnp.float32)])
    s2p = _sc_l2(srcp, dstp, s1p, dinv.reshape(-1), params)

    def hv(v):
        return v[:N].reshape(N // 5, 5)

    return _tc_head(hv(s2p[:NP]), hv(s2p[NP:]), hv(dinv.reshape(-1)),
                    b2, W3, b3)


# speedup vs baseline: 93.8616x; 1.0006x over previous
import jax
import jax.numpy as jnp
from jax import lax
from jax.experimental import pallas as pl
from jax.experimental.pallas import tpu as pltpu
from jax.experimental.pallas import tpu_sc as plsc

N = 10000
D = 128
E = 320000

NC = 2
NS = 16
NW = NC * NS
LANES = 16
NPT = 640
NP = NS * NPT
NB = NPT // LANES
EPW = E // NW
C = 128
NCHUNK = 80
EPW_PAD = NCHUNK * C
EPAD = EPW_PAD - EPW
KPC = C // LANES


def _mesh():
    return plsc.VectorSubcoreMesh(core_axis_name="c", subcore_axis_name="s")


def _zero_fill(ref):
    @pl.loop(0, NB)
    def _(b):
        ref[pl.ds(b * LANES, LANES)] = jnp.full((LANES,), 0.0, jnp.float32)


def _deg_body(dst_hbm, out_hbm, dst_v, ones_v, nodef_v, deg_sh,
              sem0, sem1):
    cid = lax.axis_index("c")
    sid = lax.axis_index("s")
    base = sid * NPT
    ebase = (cid * NS + sid) * EPW_PAD

    pltpu.sync_copy(dst_hbm.at[pl.ds(ebase, EPW_PAD)], dst_v)
    for b in range(C // LANES):
        ones_v[pl.ds(b * LANES, LANES)] = jnp.full((LANES,), 1.0, jnp.float32)
    _zero_fill(nodef_v)
    pltpu.sync_copy(nodef_v, deg_sh.at[pl.ds(base, NPT)])

    plsc.subcore_barrier()

    def dref(j):
        return deg_sh.at[dst_v.at[pl.ds(j * C, C)]]

    pltpu.async_copy(ones_v, dref(0), sem0, add=True)

    @pl.loop(0, NCHUNK // 2)
    def _(jj):
        j0 = 2 * jj
        j1 = j0 + 1

        @pl.when(jj > 0)
        def _():
            pltpu.make_async_copy(ones_v, dref(j1 - 2), sem1).wait()
        pltpu.async_copy(ones_v, dref(j1), sem1, add=True)

        @pl.when(jj < NCHUNK // 2 - 1)
        def _():
            pltpu.make_async_copy(ones_v, dref(j0), sem0).wait()
            pltpu.async_copy(ones_v, dref(j0 + 2), sem0, add=True)

    pltpu.make_async_copy(ones_v, dref(NCHUNK - 2), sem0).wait()
    pltpu.make_async_copy(ones_v, dref(NCHUNK - 1), sem1).wait()

    plsc.subcore_barrier()
    pltpu.sync_copy(deg_sh.at[pl.ds(base, NPT)],
                    out_hbm.at[pl.ds(cid * NP + base, NPT)])


@jax.jit
def _sc_deg(dstp):
    f = pl.kernel(
        _deg_body,
        out_type=jax.ShapeDtypeStruct((NC * NP,), jnp.float32),
        mesh=_mesh(),
        scratch_types=[
            pltpu.VMEM((EPW_PAD,), jnp.int32),
            pltpu.VMEM((C,), jnp.float32),
            pltpu.VMEM((NPT,), jnp.float32),
            pltpu.VMEM_SHARED((NP,), jnp.float32),
            pltpu.SemaphoreType.DMA,
            pltpu.SemaphoreType.DMA,
        ],
    )
    return f(dstp)


def _l1_body(src_hbm, dst_hbm, g1t_hbm, out_hbm,
             src_v, dst_v, msg_v, col_v,
             g1l0_v, g1l1_v, g1l2_v,
             s1c0_sh, s1c1_sh, s1c2_sh, sem0, sem1):
    g1l = [g1l0_v, g1l1_v, g1l2_v]
    s1c = [s1c0_sh, s1c1_sh, s1c2_sh]
    sems = [sem0, sem1]
    cid = lax.axis_index("c")
    sid = lax.axis_index("s")
    base = sid * NPT
    ebase = (cid * NS + sid) * EPW_PAD

    pltpu.sync_copy(src_hbm.at[pl.ds(ebase, EPW_PAD)], src_v)
    pltpu.sync_copy(dst_hbm.at[pl.ds(ebase, EPW_PAD)], dst_v)
    for c in range(3):
        pltpu.sync_copy(g1t_hbm.at[pl.ds(c * NP, NP)], g1l[c])

    @pl.when(cid == 0)
    def _():
        for c in range(3):
            pltpu.sync_copy(g1l[c].at[pl.ds(base, NPT)],
                            s1c[c].at[pl.ds(base, NPT)])

    @pl.when(cid != 0)
    def _():
        _zero_fill(col_v)
        for c in range(3):
            pltpu.sync_copy(col_v, s1c[c].at[pl.ds(base, NPT)])

    plsc.subcore_barrier()

    def build(j, p):
        for k in range(KPC):
            idx16 = src_v[pl.ds(j * C + k * LANES, LANES)]
            for c in range(3):
                msg_v[pl.ds((p * 3 + c) * C + k * LANES, LANES)] = (
                    plsc.load_gather(g1l[c], [idx16]))

    def mb(p, c):
        return msg_v.at[pl.ds((p * 3 + c) * C, C)]

    def sref(j, c):
        return s1c[c].at[dst_v.at[pl.ds(j * C, C)]]

    def sstart(j, p):
        for c in range(3):
            pltpu.async_copy(mb(p, c), sref(j, c), sems[p], add=True)

    def swait(j, p):
        for c in range(3):
            pltpu.make_async_copy(mb(p, c), sref(j, c), sems[p]).wait()

    build(0, 0)
    sstart(0, 0)

    @pl.loop(0, NCHUNK // 2)
    def _(jj):
        j0 = 2 * jj
        j1 = j0 + 1

        @pl.when(jj > 0)
        def _():
            swait(j1 - 2, 1)
        build(j1, 1)
        sstart(j1, 1)

        @pl.when(jj < NCHUNK // 2 - 1)
        def _():
            swait(j0, 0)
            build(j0 + 2, 0)
            sstart(j0 + 2, 0)

    swait(NCHUNK - 2, 0)
    swait(NCHUNK - 1, 1)

    plsc.subcore_barrier()
    for c in range(3):
        pltpu.sync_copy(
            s1c[c].at[pl.ds(base, NPT)],
            out_hbm.at[pl.ds(c * NC * NP + cid * NP + base, NPT)])


@jax.jit
def _sc_l1(srcp, dstp, g1t3):
    f = pl.kernel(
        _l1_body,
        out_type=jax.ShapeDtypeStruct((3 * NC * NP,), jnp.float32),
        mesh=_mesh(),
        compiler_params=pltpu.CompilerParams(needs_layout_passes=False),
        scratch_types=[
            pltpu.VMEM((EPW_PAD,), jnp.int32),
            pltpu.VMEM((EPW_PAD,), jnp.int32),
            pltpu.VMEM((2 * 3 * C,), jnp.float32),
            pltpu.VMEM((NPT,), jnp.float32),
            pltpu.VMEM((NP,), jnp.float32),
            pltpu.VMEM((NP,), jnp.float32),
            pltpu.VMEM((NP,), jnp.float32),
            pltpu.VMEM_SHARED((NP,), jnp.float32),
            pltpu.VMEM_SHARED((NP,), jnp.float32),
            pltpu.VMEM_SHARED((NP,), jnp.float32),
            pltpu.SemaphoreType.DMA,
            pltpu.SemaphoreType.DMA,
        ],
    )
    return f(srcp, dstp, g1t3)


def _l2_body(src_hbm, dst_hbm, s1_hbm, dinv_hbm, params_hbm, out_hbm,
             src_v, dst_v, msg_v, col_v, g2l_v, params_v, dinv_v,
             colsa_v, colsb_v, vec_v, g2st_hbm, s2_sh, sem0, sem1):
    sems = [sem0, sem1]
    cid = lax.axis_index("c")
    sid = lax.axis_index("s")
    base = sid * NPT
    ebase = (cid * NS + sid) * EPW_PAD

    pltpu.sync_copy(src_hbm.at[pl.ds(ebase, EPW_PAD)], src_v)
    pltpu.sync_copy(dst_hbm.at[pl.ds(ebase, EPW_PAD)], dst_v)
    pltpu.sync_copy(params_hbm, params_v)
    pltpu.sync_copy(dinv_hbm.at[pl.ds(base, NPT)], dinv_v)
    for c in range(3):
        pltpu.sync_copy(s1_hbm.at[pl.ds(c * NC * NP + base, NPT)],
                        colsa_v.at[pl.ds(c * NPT, NPT)])
        pltpu.sync_copy(s1_hbm.at[pl.ds(c * NC * NP + NP + base, NPT)],
                        colsb_v.at[pl.ds(c * NPT, NPT)])

    @pl.loop(0, NB)
    def _(b):
        bs = pl.ds(b * LANES, LANES)
        dinv16 = dinv_v[bs]
        h2 = jnp.full((LANES,), 0.0, jnp.float32)
        for c in range(3):
            cs = pl.ds(c * NPT + b * LANES, LANES)
            b1c = params_v[pl.ds(c * LANES, LANES)]
            w2c = params_v[pl.ds((3 + c) * LANES, LANES)]
            a1c = jnp.maximum(dinv16 * (colsa_v[cs] + colsb_v[cs]) + b1c, 0.0)
            h2 = h2 + a1c * w2c
        vec_v[bs] = dinv16 * h2

    pltpu.sync_copy(vec_v, g2st_hbm.at[pl.ds(cid * NP + base, NPT)])

    @pl.when(cid == 0)
    def _():
        pltpu.sync_copy(vec_v, s2_sh.at[pl.ds(base, NPT)])

    @pl.when(cid != 0)
    def _():
        _zero_fill(col_v)
        pltpu.sync_copy(col_v, s2_sh.at[pl.ds(base, NPT)])

    plsc.subcore_barrier()
    pltpu.sync_copy(g2st_hbm.at[pl.ds(cid * NP, NP)], g2l_v)

    def build(j, p):
        for k in range(KPC):
            idx16 = src_v[pl.ds(j * C + k * LANES, LANES)]
            msg_v[pl.ds(p * C + k * LANES, LANES)] = (
                plsc.load_gather(g2l_v, [idx16]))

    def mb(p):
        return msg_v.at[pl.ds(p * C, C)]

    def sref(j):
        return s2_sh.at[dst_v.at[pl.ds(j * C, C)]]

    build(0, 0)
    pltpu.async_copy(mb(0), sref(0), sem0, add=True)

    @pl.loop(0, NCHUNK // 2)
    def _(jj):
        j0 = 2 * jj
        j1 = j0 + 1

        @pl.when(jj > 0)
        def _():
            pltpu.make_async_copy(mb(1), sref(j1 - 2), sem1).wait()
        build(j1, 1)
        pltpu.async_copy(mb(1), sref(j1), sem1, add=True)

        @pl.when(jj < NCHUNK // 2 - 1)
        def _():
            pltpu.make_async_copy(mb(0), sref(j0), sem0).wait()
            build(j0 + 2, 0)
            pltpu.async_copy(mb(0), sref(j0 + 2), sem0, add=True)

    pltpu.make_async_copy(mb(0), sref(NCHUNK - 2), sem0).wait()
    pltpu.make_async_copy(mb(1), sref(NCHUNK - 1), sem1).wait()

    plsc.subcore_barrier()
    pltpu.sync_copy(s2_sh.at[pl.ds(base, NPT)],
                    out_hbm.at[pl.ds(cid * NP + base, NPT)])


@jax.jit
def _sc_l2(srcp, dstp, s1p, dinv, params):
    f = pl.kernel(
        _l2_body,
        out_type=jax.ShapeDtypeStruct((NC * NP,), jnp.float32),
        mesh=_mesh(),
        compiler_params=pltpu.CompilerParams(needs_layout_passes=False),
        scratch_types=[
            pltpu.VMEM((EPW_PAD,), jnp.int32),
            pltpu.VMEM((EPW_PAD,), jnp.int32),
            pltpu.VMEM((2 * C,), jnp.float32),
            pltpu.VMEM((NPT,), jnp.float32),
            pltpu.VMEM((NP,), jnp.float32),
            pltpu.VMEM((8 * LANES,), jnp.float32),
            pltpu.VMEM((NPT,), jnp.float32),
            pltpu.VMEM((3 * NPT,), jnp.float32),
            pltpu.VMEM((3 * NPT,), jnp.float32),
            pltpu.VMEM((NPT,), jnp.float32),
            pltpu.HBM((NC * NP,), jnp.float32),
            pltpu.VMEM_SHARED((NP,), jnp.float32),
            pltpu.SemaphoreType.DMA,
            pltpu.SemaphoreType.DMA,
        ],
    )
    return f(srcp, dstp, s1p, dinv, params)


def _tc1_body(x_ref, w_ref, da_ref, db_ref, g1t_ref, dinv_ref):
    h1 = jnp.dot(x_ref[...], w_ref[...], preferred_element_type=jnp.float32)
    deg = 1.0 + da_ref[...] + db_ref[...]
    dinv = lax.rsqrt(deg)
    g1t_ref[...] = h1.T * dinv
    dinv_ref[...] = dinv


@jax.jit
def _tc_prep(xp, W1, dega, degb):
    blk = 1024
    return pl.pallas_call(
        _tc1_body,
        grid=(NP // blk,),
        in_specs=[
            pl.BlockSpec((blk, D), lambda i: (i, 0)),
            pl.BlockSpec((D, 4), lambda i: (0, 0)),
            pl.BlockSpec((1, blk), lambda i: (0, i)),
            pl.BlockSpec((1, blk), lambda i: (0, i)),
        ],
        out_specs=[
            pl.BlockSpec((4, blk), lambda i: (0, i)),
            pl.BlockSpec((1, blk), lambda i: (0, i)),
        ],
        out_shape=[
            jax.ShapeDtypeStruct((4, NP), jnp.float32),
            jax.ShapeDtypeStruct((1, NP), jnp.float32),
        ],
    )(xp, W1, dega, degb)


def _tc3_body(sa_ref, sb_ref, dinv_ref, b2_ref, w3_ref, b3_ref, o_ref):
    out2 = dinv_ref[...] * (sa_ref[...] + sb_ref[...]) + b2_ref[0, 0]
    logits = jnp.dot(out2, w3_ref[...], preferred_element_type=jnp.float32)
    logits = logits + b3_ref[...]
    m = jnp.max(logits, axis=1, keepdims=True)
    e = jnp.exp(logits - m)
    lse = m + jnp.log(jnp.sum(e, axis=1, keepdims=True))
    o_ref[...] = logits - lse


@jax.jit
def _tc_head(s2a, s2b, dinv, b2, W3, b3):
    return pl.pallas_call(
        _tc3_body,
        out_shape=jax.ShapeDtypeStruct((N // 5, 2), jnp.float32),
    )(s2a, s2b, dinv, b2.reshape(1, 1), W3, b3.reshape(1, 2))


def kernel(x, edge_index, W1, b1, W2, b2, W3, b3):
    pad_rows = N + jnp.arange(EPAD, dtype=jnp.int32)
    src2 = edge_index[0].reshape(NW, EPW)
    dst2 = edge_index[1].reshape(NW, EPW)
    pad2 = jnp.broadcast_to(pad_rows, (NW, EPAD))
    srcp = jnp.concatenate([src2, pad2], axis=1).reshape(-1)
    dstp = jnp.concatenate([dst2, pad2], axis=1).reshape(-1)

    degp = _sc_deg(dstp)
    dega = degp[:NP].reshape(1, NP)
    degb = degp[NP:].reshape(1, NP)

    xp = jnp.pad(x, ((0, NP - N), (0, 0)))
    W1p = jnp.pad(W1, ((0, 0), (0, 1)))
    g1t, dinv = _tc_prep(xp, W1p, dega, degb)

    s1p = _sc_l1(srcp, dstp, g1t[:3].reshape(-1))

    params = jnp.concatenate(
        [jnp.repeat(b1, LANES), jnp.repeat(W2[:, 0], LANES),
         jnp.zeros((2 * LANES,), jnp.float32)])
    s2p = _sc_l2(srcp, dstp, s1p, dinv.reshape(-1), params)

    def hv(v):
        return v[:N].reshape(N // 5, 5)

    return _tc_head(hv(s2p[:NP]), hv(s2p[NP:]), hv(dinv.reshape(-1)),
                    b2, W3, b3)
